# baseline (speedup 1.0000x reference)
import functools

import jax
import jax.numpy as jnp
import numpy as np
from jax import lax
from jax.experimental import pallas as pl
from jax.experimental.pallas import tpu as pltpu
from jax.experimental.pallas import tpu_sc as plsc

H = 128
N_REAL = 10000
N_PAD = 10240
N_ACC = 10112
NC_PAD = 512
DUMMY = N_ACC - 1
CHW = 80
SCH = 8 * CHW
EP_PAD = 163840
EV_PAD = 327680
EC_PAD = 327680
EC_REAL = 320000
NW = 32
BLK = 1024
P_STEPS = 3
V_STEPS = 4

_HIGH = jax.lax.Precision.HIGHEST


def _dot(a, b):
    return lax.dot_general(a, b, (((1,), (0,)), ((), ())),
                           preferred_element_type=jnp.float32,
                           precision=_HIGH)


def _dott(a, b):
    return lax.dot_general(a, b, (((0,), (0,)), ((), ())),
                           preferred_element_type=jnp.float32,
                           precision=_HIGH)


def _fused_rows(n, terms, bias=None, act="none", residual=None, out_dim=H):
    nb = n // BLK
    args, specs = [], []

    def add(a, spec):
        args.append(a)
        specs.append(spec)

    for t in terms:
        kind = t[0]
        if kind == "mm":
            _, X, W = t
            add(X, pl.BlockSpec((BLK, X.shape[1]), lambda i: (i, 0)))
            add(W, pl.BlockSpec(W.shape, lambda i: (0, 0)))
        elif kind == "parts":
            _, P, dv, W = t
            add(P, pl.BlockSpec((2, BLK, H), lambda i: (0, i, 0)))
            if dv is not None:
                add(dv, pl.BlockSpec((BLK, 1), lambda i: (i, 0)))
            add(W, pl.BlockSpec(W.shape, lambda i: (0, 0)))
        elif kind == "onehot":
            _, ids3, T, sc = t
            add(ids3, pl.BlockSpec((1, 1, BLK), lambda i: (i, 0, 0)))
            add(T, pl.BlockSpec(T.shape, lambda i: (0, 0)))
            if sc is not None:
                add(sc, pl.BlockSpec((BLK, 1), lambda i: (i, 0)))
        elif kind == "gparts":
            _, P, sc, S = t
            add(P, pl.BlockSpec((2, BLK, out_dim), lambda i: (0, i, 0)))
            add(sc, pl.BlockSpec((BLK, 1), lambda i: (i, 0)))
            add(S, pl.BlockSpec((1, 1), lambda i: (0, 0)))
    if bias is not None:
        add(bias.reshape(1, out_dim), pl.BlockSpec((1, out_dim), lambda i: (0, 0)))
    if residual is not None:
        add(residual, pl.BlockSpec((BLK, out_dim), lambda i: (i, 0)))

    def body(*refs):
        refs = list(refs)
        out_ref = refs.pop()
        acc = jnp.zeros((BLK, out_dim), jnp.float32)
        for t in terms:
            kind = t[0]
            if kind == "mm":
                xr = refs.pop(0)[...]
                wr = refs.pop(0)[...]
                acc += _dot(xr, wr)
            elif kind == "parts":
                pr = refs.pop(0)[...]
                p = pr[0] + pr[1]
                if t[2] is not None:
                    p = p / refs.pop(0)[...]
                acc += _dot(p, refs.pop(0)[...])
            elif kind == "onehot":
                ids = refs.pop(0)[0, 0, :]
                T = refs.pop(0)[...]
                C = T.shape[0]
                oh = (ids[:, None] ==
                      lax.broadcasted_iota(jnp.int32, (BLK, C), 1)).astype(jnp.float32)
                g = _dot(oh, T)
                if t[3] is not None:
                    g = g * refs.pop(0)[...]
                acc += g
            elif kind == "gparts":
                pr = refs.pop(0)[...]
                scr = refs.pop(0)[...]
                Sr = refs.pop(0)[0, 0]
                acc += (pr[0] + pr[1]) * (scr / Sr)
        if bias is not None:
            acc += refs.pop(0)[...]
        if act == "lrelu":
            acc = jnp.maximum(acc, 0.01 * acc)
        elif act == "sigmoid":
            acc = jax.nn.sigmoid(acc)
        if residual is not None:
            acc += refs.pop(0)[...]
        out_ref[...] = acc

    return pl.pallas_call(
        body,
        grid=(nb,),
        in_specs=specs,
        out_specs=pl.BlockSpec((BLK, out_dim), lambda i: (i, 0)),
        out_shape=jax.ShapeDtypeStruct((n, out_dim), jnp.float32),
    )(*args)


def _cluster_sum(x, ids3):
    def body(ids_ref, x_ref, out_ref):
        @pl.when(pl.program_id(0) == 0)
        def _():
            out_ref[...] = jnp.zeros_like(out_ref)
        ids = ids_ref[0, 0, :]
        oh = (ids[:, None] ==
              lax.broadcasted_iota(jnp.int32, (BLK, NC_PAD), 1)).astype(jnp.float32)
        out_ref[...] += _dott(oh, x_ref[...])

    return pl.pallas_call(
        body,
        grid=(N_PAD // BLK,),
        in_specs=[pl.BlockSpec((1, 1, BLK), lambda i: (i, 0, 0)),
                  pl.BlockSpec((BLK, H), lambda i: (i, 0))],
        out_specs=pl.BlockSpec((NC_PAD, H), lambda i: (0, 0)),
        out_shape=jax.ShapeDtypeStruct((NC_PAD, H), jnp.float32),
    )(ids3, x)


def _cluster_counts(ids3):
    def body(ids_ref, out_ref):
        @pl.when(pl.program_id(0) == 0)
        def _():
            out_ref[...] = jnp.zeros_like(out_ref)
        ids = ids_ref[0, 0, :]
        oh = (ids[:, None] ==
              lax.broadcasted_iota(jnp.int32, (BLK, NC_PAD), 1)).astype(jnp.float32)
        out_ref[...] += _dott(oh, jnp.ones((BLK, 1), jnp.float32))

    return pl.pallas_call(
        body,
        grid=(N_PAD // BLK,),
        in_specs=[pl.BlockSpec((1, 1, BLK), lambda i: (i, 0, 0))],
        out_specs=pl.BlockSpec((NC_PAD, 1), lambda i: (0, 0)),
        out_shape=jax.ShapeDtypeStruct((NC_PAD, 1), jnp.float32),
    )(ids3)


def _cm_proj(csum, ccnt, U3):
    def body(cs_ref, cc_ref, u_ref, out_ref):
        cm = cs_ref[...] / jnp.maximum(cc_ref[...], 1.0)
        out_ref[...] = _dot(cm, u_ref[...])

    return pl.pallas_call(
        body,
        out_shape=jax.ShapeDtypeStruct((NC_PAD, H), jnp.float32),
    )(csum, ccnt, U3)


def _deg_reduce(parts):
    def body(p_ref, out_ref):
        p = p_ref[...]
        out_ref[...] = jnp.maximum(p[0, :, :1] + p[1, :, :1], 1.0)

    return pl.pallas_call(
        body,
        grid=(N_PAD // BLK,),
        in_specs=[pl.BlockSpec((2, BLK, 16), lambda i: (0, i, 0))],
        out_specs=pl.BlockSpec((BLK, 1), lambda i: (i, 0)),
        out_shape=jax.ShapeDtypeStruct((N_PAD, 1), jnp.float32),
    )(parts)


_ZROWS = EC_PAD // 128
_ZBLK = 16


def _ptr_phase_a(zp, u2d):
    nb = _ZROWS // _ZBLK

    def body(zp_ref, u_ref, z_ref, m_ref):
        i = pl.program_id(0)
        s = jnp.sum(zp_ref[...], axis=1).reshape(_ZBLK, 128)
        u = u_ref[...]
        g = -jnp.log(-jnp.log(u))
        ids = (i * (_ZBLK * 128) +
               lax.broadcasted_iota(jnp.int32, (_ZBLK, 128), 0) * 128 +
               lax.broadcasted_iota(jnp.int32, (_ZBLK, 128), 1))
        z = jnp.where(ids < EC_REAL, s + g, -1e30)
        z_ref[...] = z

        @pl.when(i == 0)
        def _():
            m_ref[...] = jnp.full((1, 1), -1e30, jnp.float32)
        m_ref[...] = jnp.maximum(m_ref[...], jnp.max(z))

    return pl.pallas_call(
        body,
        grid=(nb,),
        in_specs=[pl.BlockSpec((_ZBLK * 128, 16), lambda i: (i, 0)),
                  pl.BlockSpec((_ZBLK, 128), lambda i: (i, 0))],
        out_specs=[pl.BlockSpec((_ZBLK, 128), lambda i: (i, 0)),
                   pl.BlockSpec((1, 1), lambda i: (0, 0))],
        out_shape=[jax.ShapeDtypeStruct((_ZROWS, 128), jnp.float32),
                   jax.ShapeDtypeStruct((1, 1), jnp.float32)],
    )(zp, u2d)


def _ptr_phase_b(z2d, m):
    nb = _ZROWS // _ZBLK

    def body(z_ref, m_ref, y_ref, s_ref):
        i = pl.program_id(0)
        y = jnp.exp(z_ref[...] - m_ref[0, 0])
        y_ref[...] = y

        @pl.when(i == 0)
        def _():
            s_ref[...] = jnp.zeros((1, 1), jnp.float32)
        s_ref[...] += jnp.sum(y)

    return pl.pallas_call(
        body,
        grid=(nb,),
        in_specs=[pl.BlockSpec((_ZBLK, 128), lambda i: (i, 0)),
                  pl.BlockSpec((1, 1), lambda i: (0, 0))],
        out_specs=[pl.BlockSpec((_ZBLK, 128), lambda i: (i, 0)),
                   pl.BlockSpec((1, 1), lambda i: (0, 0))],
        out_shape=[jax.ShapeDtypeStruct((_ZROWS, 128), jnp.float32),
                   jax.ShapeDtypeStruct((1, 1), jnp.float32)],
    )(z2d, m)


_MESH = plsc.VectorSubcoreMesh(core_axis_name="c", subcore_axis_name="s")
_ZR = N_ACC // 16
_NTAIL = N_PAD - N_ACC


def _sc_worker_ids():
    c = lax.axis_index("c")
    s = lax.axis_index("s")
    return c, s, s * 2 + c


def _sc_edge_aggr(A, B, src3, dst3, zeros_big, nq):
    @functools.partial(
        pl.kernel,
        out_type=jax.ShapeDtypeStruct((2, N_PAD, H), jnp.float32),
        mesh=_MESH,
        scratch_types=[
            pltpu.VMEM((8, CHW), jnp.int32),
            pltpu.VMEM((8, CHW), jnp.int32),
            pltpu.VMEM((2, CHW, H), jnp.float32),
            pltpu.VMEM((2, CHW, H), jnp.float32),
            pltpu.VMEM_SHARED((N_ACC, H), jnp.float32),
            pltpu.SemaphoreType.DMA,
            pltpu.SemaphoreType.DMA,
            pltpu.SemaphoreType.DMA,
            pltpu.SemaphoreType.DMA,
            pltpu.SemaphoreType.DMA,
            pltpu.SemaphoreType.DMA,
        ],
    )
    def k(A_hbm, B_hbm, src_hbm, dst_hbm, z_hbm, out_hbm,
          sidx, didx, rowsA, rowsB, acc,
          semA0, semA1, semB0, semB1, semS0, semS1):
        c, s, wid = _sc_worker_ids()
        base = wid * nq
        semA = (semA0, semA1)
        semB = (semB0, semB1)
        semS = (semS0, semS1)
        pltpu.sync_copy(z_hbm.at[pl.ds(s * _ZR, _ZR)], acc.at[pl.ds(s * _ZR, _ZR)])
        plsc.subcore_barrier()

        def gather(cc, p):
            ga = pltpu.async_copy(A_hbm.at[didx.at[cc]], rowsA.at[p], semA[p])
            gb = pltpu.async_copy(B_hbm.at[sidx.at[cc]], rowsB.at[p], semB[p])
            return ga, gb

        def compute(p):
            def row(r, carry2):
                for kk in range(8):
                    a = (rowsA[p, r, pl.ds(kk * 16, 16)] +
                         rowsB[p, r, pl.ds(kk * 16, 16)])
                    rowsA[p, r, pl.ds(kk * 16, 16)] = jnp.maximum(a, 0.01 * a)
                return carry2
            lax.fori_loop(0, CHW, row, 0, unroll=4)

        def superchunk(q, carry):
            pltpu.sync_copy(src_hbm.at[base + q], sidx)
            pltpu.sync_copy(dst_hbm.at[base + q], didx)
            g = [None, None]
            sc_pend = [None, None]
            g[0] = gather(0, 0)
            for cc in range(8):
                p = cc & 1
                g[p][0].wait()
                g[p][1].wait()
                if cc < 7:
                    if sc_pend[1 - p] is not None:
                        sc_pend[1 - p].wait()
                        sc_pend[1 - p] = None
                    g[1 - p] = gather(cc + 1, 1 - p)
                compute(p)
                sc_pend[p] = pltpu.async_copy(
                    rowsA.at[p], acc.at[didx.at[cc]], semS[p], add=True)
            sc_pend[0].wait()
            sc_pend[1].wait()
            return carry

        lax.fori_loop(0, nq, superchunk, 0)
        plsc.subcore_barrier()
        pltpu.sync_copy(acc.at[pl.ds(s * _ZR, _ZR)],
                        out_hbm.at[c, pl.ds(s * _ZR, _ZR)])

        @pl.when(s == 0)
        def _():
            pltpu.sync_copy(z_hbm.at[pl.ds(0, _NTAIL)],
                            out_hbm.at[c, pl.ds(N_ACC, _NTAIL)])

    return k(A, B, src3, dst3, zeros_big)


def _sc_degree(dst3, ones16, zeros16, nq):
    @functools.partial(
        pl.kernel,
        out_type=jax.ShapeDtypeStruct((2, N_PAD, 16), jnp.float32),
        mesh=_MESH,
        scratch_types=[
            pltpu.VMEM((8, CHW), jnp.int32),
            pltpu.VMEM((CHW, 16), jnp.float32),
            pltpu.VMEM_SHARED((N_ACC, 16), jnp.float32),
            pltpu.SemaphoreType.DMA,
        ],
    )
    def k(dst_hbm, ones_hbm, z_hbm, out_hbm, didx, ones_v, acc, semS):
        c, s, wid = _sc_worker_ids()
        base = wid * nq
        pltpu.sync_copy(ones_hbm, ones_v)
        pltpu.sync_copy(z_hbm.at[pl.ds(s * _ZR, _ZR), pl.ds(0, 16)],
                        acc.at[pl.ds(s * _ZR, _ZR)])
        plsc.subcore_barrier()

        def superchunk(q, carry):
            pltpu.sync_copy(dst_hbm.at[base + q], didx)
            pend = []
            for cc in range(8):
                pend.append(pltpu.async_copy(
                    ones_v, acc.at[didx.at[cc]], semS, add=True))
            for d in pend:
                d.wait()
            return carry

        lax.fori_loop(0, nq, superchunk, 0)
        plsc.subcore_barrier()
        pltpu.sync_copy(acc.at[pl.ds(s * _ZR, _ZR)],
                        out_hbm.at[c, pl.ds(s * _ZR, _ZR)])

        @pl.when(s == 0)
        def _():
            pltpu.sync_copy(z_hbm.at[pl.ds(0, _NTAIL), pl.ds(0, 16)],
                            out_hbm.at[c, pl.ds(N_ACC, _NTAIL)])

    return k(dst3, ones16, zeros16)


def _sc_ptr_score(Xp, Vv, ce0_3d, ce1_3d, theta816, nq):
    @functools.partial(
        pl.kernel,
        out_type=jax.ShapeDtypeStruct((EC_PAD // CHW, CHW, 16), jnp.float32),
        mesh=_MESH,
        scratch_types=[
            pltpu.VMEM((8, CHW), jnp.int32),
            pltpu.VMEM((8, CHW), jnp.int32),
            pltpu.VMEM((2, CHW, H), jnp.float32),
            pltpu.VMEM((2, CHW, H), jnp.float32),
            pltpu.VMEM((2, CHW, 16), jnp.float32),
            pltpu.VMEM((8, 16), jnp.float32),
            pltpu.SemaphoreType.DMA,
            pltpu.SemaphoreType.DMA,
            pltpu.SemaphoreType.DMA,
            pltpu.SemaphoreType.DMA,
            pltpu.SemaphoreType.DMA,
            pltpu.SemaphoreType.DMA,
        ],
    )
    def k(Xp_hbm, Vv_hbm, i0_hbm, i1_hbm, th_hbm, out_hbm,
          i0, i1, rowsX, rowsV, zbuf, thv,
          semA0, semA1, semB0, semB1, semS0, semS1):
        c, s, wid = _sc_worker_ids()
        base = wid * nq
        semA = (semA0, semA1)
        semB = (semB0, semB1)
        semS = (semS0, semS1)
        pltpu.sync_copy(th_hbm, thv)

        def gather(cc, p):
            ga = pltpu.async_copy(Xp_hbm.at[i0.at[cc]], rowsX.at[p], semA[p])
            gb = pltpu.async_copy(Vv_hbm.at[i1.at[cc]], rowsV.at[p], semB[p])
            return ga, gb

        def compute(p):
            th = [thv[kk, :] for kk in range(8)]

            def row(r, carry2):
                acc = jnp.zeros((16,), jnp.float32)
                for kk in range(8):
                    zz = (rowsX[p, r, pl.ds(kk * 16, 16)] +
                          rowsV[p, r, pl.ds(kk * 16, 16)])
                    t = 1.0 - 2.0 / (jnp.exp(2.0 * zz) + 1.0)
                    acc = acc + th[kk] * t
                zbuf[p, r, :] = acc
                return carry2
            lax.fori_loop(0, CHW, row, 0, unroll=2)

        def superchunk(q, carry):
            pltpu.sync_copy(i0_hbm.at[base + q], i0)
            pltpu.sync_copy(i1_hbm.at[base + q], i1)
            g = [None, None]
            st_pend = [None, None]
            g[0] = gather(0, 0)
            for cc in range(8):
                p = cc & 1
                g[p][0].wait()
                g[p][1].wait()
                if cc < 7:
                    g[1 - p] = gather(cc + 1, 1 - p)
                if st_pend[p] is not None:
                    st_pend[p].wait()
                    st_pend[p] = None
                compute(p)
                st_pend[p] = pltpu.async_copy(
                    zbuf.at[p], out_hbm.at[(base + q) * 8 + cc], semS[p])
            st_pend[0].wait()
            st_pend[1].wait()
            return carry

        lax.fori_loop(0, nq, superchunk, 0)

    return k(Xp, Vv, ce0_3d, ce1_3d, theta816)


def _sc_ptr_scatter(x, ce0_3d, ce1_3d, y3d, zeros_big, nq):
    @functools.partial(
        pl.kernel,
        out_type=jax.ShapeDtypeStruct((2, N_PAD, H), jnp.float32),
        mesh=_MESH,
        scratch_types=[
            pltpu.VMEM((8, CHW), jnp.int32),
            pltpu.VMEM((8, CHW), jnp.int32),
            pltpu.VMEM((2, CHW, H), jnp.float32),
            pltpu.VMEM((8, CHW), jnp.float32),
            pltpu.VMEM_SHARED((N_ACC, H), jnp.float32),
            pltpu.SemaphoreType.DMA,
            pltpu.SemaphoreType.DMA,
            pltpu.SemaphoreType.DMA,
            pltpu.SemaphoreType.DMA,
        ],
    )
    def k(x_hbm, i0_hbm, i1_hbm, y_hbm, z_hbm, out_hbm,
          i0, i1, rowsX, ybuf, acc, semA0, semA1, semS0, semS1):
        c, s, wid = _sc_worker_ids()
        base = wid * nq
        semA = (semA0, semA1)
        semS = (semS0, semS1)
        pltpu.sync_copy(z_hbm.at[pl.ds(s * _ZR, _ZR)], acc.at[pl.ds(s * _ZR, _ZR)])
        plsc.subcore_barrier()

        def compute(p, cc):
            def grp(gg, carry2):
                yvec = ybuf[cc, pl.ds(gg * 16, 16)]
                for lane in range(16):
                    r = gg * 16 + lane
                    yv = yvec[lane]
                    for kk in range(8):
                        rowsX[p, r, pl.ds(kk * 16, 16)] = (
                            rowsX[p, r, pl.ds(kk * 16, 16)] * yv)
                return carry2
            lax.fori_loop(0, CHW // 16, grp, 0, unroll=2)

        def superchunk(q, carry):
            pltpu.sync_copy(i0_hbm.at[base + q], i0)
            pltpu.sync_copy(i1_hbm.at[base + q], i1)
            pltpu.sync_copy(y_hbm.at[base + q], ybuf)
            g = [None, None]
            sc_pend = [None, None]
            g[0] = pltpu.async_copy(x_hbm.at[i0.at[0]], rowsX.at[0], semA[0])
            for cc in range(8):
                p = cc & 1
                g[p].wait()
                if cc < 7:
                    if sc_pend[1 - p] is not None:
                        sc_pend[1 - p].wait()
                        sc_pend[1 - p] = None
                    g[1 - p] = pltpu.async_copy(
                        x_hbm.at[i0.at[cc + 1]], rowsX.at[1 - p], semA[1 - p])
                compute(p, cc)
                sc_pend[p] = pltpu.async_copy(
                    rowsX.at[p], acc.at[i1.at[cc]], semS[p], add=True)
            sc_pend[0].wait()
            sc_pend[1].wait()
            return carry

        lax.fori_loop(0, nq, superchunk, 0)
        plsc.subcore_barrier()
        pltpu.sync_copy(acc.at[pl.ds(s * _ZR, _ZR)],
                        out_hbm.at[c, pl.ds(s * _ZR, _ZR)])

        @pl.when(s == 0)
        def _():
            pltpu.sync_copy(z_hbm.at[pl.ds(0, _NTAIL)],
                            out_hbm.at[c, pl.ds(N_ACC, _NTAIL)])

    return k(x, ce0_3d, ce1_3d, y3d, zeros_big)


def _pad_rows(a, n):
    return jnp.pad(a, ((0, n - a.shape[0]), (0, 0)))


def _prep_edges(ei, epad):
    e = ei.shape[1]
    p = jnp.pad(ei, ((0, 0), (0, epad - e)), constant_values=DUMMY)
    return (p[0].reshape(epad // SCH, 8, CHW), p[1].reshape(epad // SCH, 8, CHW))


def _pe_host():
    pos = np.arange(100, dtype=np.float32)[:, None]
    i2 = np.arange(0, H, 2, dtype=np.float32)
    ang = pos / (10000.0 ** (i2 / H))
    t = np.zeros((128, H), dtype=np.float32)
    t[:100, 0::2] = np.sin(ang)
    t[:100, 1::2] = np.cos(ang)
    return jnp.asarray(t)


def kernel(local_x, local_edge_index, node_cluster, node_ratio, voxel_x,
           voxel_edge_index, voxel_level, cross_edge_index, program_noise,
           voxel_noise, params):
    nq_p = EP_PAD // NW // SCH
    nq_v = EV_PAD // NW // SCH
    src_p, dst_p = _prep_edges(local_edge_index, EP_PAD)
    src_v, dst_v = _prep_edges(voxel_edge_index, EV_PAD)
    ce0, ce1 = _prep_edges(cross_edge_index, EC_PAD)

    ids3 = jnp.pad(node_cluster, (0, N_PAD - N_REAL),
                   constant_values=NC_PAD - 1).reshape(N_PAD // BLK, 1, BLK)
    lvl3 = jnp.pad(voxel_level, (0, N_PAD - N_REAL)).reshape(N_PAD // BLK, 1, BLK)
    ratio = _pad_rows(jnp.sum(node_ratio, axis=1)[:, None], N_PAD)

    zeros_big = jnp.zeros((N_PAD, H), jnp.float32)
    zeros16 = jnp.zeros((N_PAD, 16), jnp.float32)
    ones16 = jnp.ones((CHW, 16), jnp.float32)
    pe_pad = _pe_host()

    u2d = {}
    for li in (1, 3):
        u = jax.random.uniform(jax.random.fold_in(jax.random.key(42), li),
                               (EC_REAL,), minval=1e-9, maxval=1.0,
                               dtype=jnp.float32)
        u2d[li] = jnp.pad(u, (0, EC_PAD - EC_REAL),
                          constant_values=0.5).reshape(_ZROWS, 128)

    degp = _sc_degree(dst_p, ones16, zeros16, nq_p)
    deg = _deg_reduce(degp)
    ccnt = _cluster_counts(ids3)

    pW, pb = params["p_enc"]["W"], params["p_enc"]["b"]
    x = _fused_rows(N_PAD,
                    [("mm", _pad_rows(local_x, N_PAD), pW[:128]),
                     ("mm", _pad_rows(program_noise, N_PAD), pW[128:])],
                    bias=pb, act="lrelu")
    pos = _fused_rows(N_PAD, [("onehot", lvl3, pe_pad, None)])
    vW, vb = params["v_enc"]["W"], params["v_enc"]["b"]
    v = _fused_rows(N_PAD,
                    [("mm", _pad_rows(voxel_x, N_PAD), vW[:128]),
                     ("mm", _pad_rows(voxel_noise, N_PAD), vW[128:])],
                    bias=vb, act="lrelu", residual=pos)

    for l in range(P_STEPS):
        W, b = params["p_msg"][l]["W"], params["p_msg"][l]["b"]
        A = _fused_rows(N_PAD, [("mm", x, W[:128])], bias=b)
        B = _fused_rows(N_PAD, [("mm", x, W[128:])])
        csum = _cluster_sum(x, ids3)
        U, ub = params["p_upd"][l]["W"], params["p_upd"][l]["b"]
        CU = _cm_proj(csum, ccnt, U[256:])
        parts = _sc_edge_aggr(A, B, src_p, dst_p, zeros_big, nq_p)
        x = _fused_rows(N_PAD,
                        [("mm", x, U[:128]),
                         ("parts", parts, deg, U[128:256]),
                         ("onehot", ids3, CU, ratio)],
                        bias=ub, act="lrelu", residual=x)

    ptr = params["ptr"]
    Xp = _fused_rows(N_PAD, [("mm", x, ptr["Wp"]["W"])], bias=ptr["Wp"]["b"])
    theta816 = ptr["theta"][:, 0].reshape(8, 16)

    for li in range(V_STEPS):
        W, b = params["v_msg"][li]["W"], params["v_msg"][li]["b"]
        A = _fused_rows(N_PAD, [("mm", v, W[:128]), ("mm", pos, W[256:])], bias=b)
        B = _fused_rows(N_PAD, [("mm", v, W[128:256]), ("mm", pos, -W[256:])])
        parts = _sc_edge_aggr(A, B, src_v, dst_v, zeros_big, nq_v)
        U, ub = params["v_upd"][li]["W"], params["v_upd"][li]["b"]
        v = _fused_rows(N_PAD,
                        [("mm", v, U[:128]), ("parts", parts, None, U[128:])],
                        bias=ub, act="lrelu", residual=v)
        if (li + 1) % 2 == 0:
            Vv = _fused_rows(N_PAD, [("mm", v, ptr["Wv"]["W"])],
                             bias=ptr["Wv"]["b"])
            h = _fused_rows(N_PAD, [("mm", v, ptr["m1"]["W"])],
                            bias=ptr["m1"]["b"], act="lrelu")
            mask = _fused_rows(N_PAD, [("mm", h, ptr["m2"]["W"])],
                               bias=ptr["m2"]["b"], act="sigmoid", out_dim=1)
            zp = _sc_ptr_score(Xp, Vv, ce0, ce1, theta816, nq_v)
            z2d, m = _ptr_phase_a(zp.reshape(EC_PAD, 16), u2d[li])
            y2d, S = _ptr_phase_b(z2d, m)
            y3d = y2d.reshape(EC_PAD // SCH, 8, CHW)
            parts = _sc_ptr_scatter(x, ce0, ce1, y3d, zeros_big, nq_v)
            v = _fused_rows(N_PAD, [("gparts", parts, mask, S)], residual=v)

    return v[:N_REAL]


# speedup vs baseline: 1.1181x; 1.1181x over previous
import functools

import jax
import jax.numpy as jnp
import numpy as np
from jax import lax
from jax.experimental import pallas as pl
from jax.experimental.pallas import tpu as pltpu
from jax.experimental.pallas import tpu_sc as plsc

H = 128
N_REAL = 10000
N_PAD = 10240
N_ACC = 10112
NC_PAD = 512
DUMMY = N_ACC - 1
CHW = 80
SCH = 8 * CHW
EP_PAD = 163840
EV_PAD = 327680
EC_PAD = 327680
EC_REAL = 320000
NW = 32
BLK = 1024
P_STEPS = 3
V_STEPS = 4

_HIGH = jax.lax.Precision.HIGHEST


def _dot(a, b):
    return lax.dot_general(a, b, (((1,), (0,)), ((), ())),
                           preferred_element_type=jnp.float32,
                           precision=_HIGH)


def _dott(a, b):
    return lax.dot_general(a, b, (((0,), (0,)), ((), ())),
                           preferred_element_type=jnp.float32,
                           precision=_HIGH)


def _fused_rows(n, terms, bias=None, act="none", residual=None, out_dim=H):
    nb = n // BLK
    args, specs = [], []

    def add(a, spec):
        args.append(a)
        specs.append(spec)

    for t in terms:
        kind = t[0]
        if kind == "mm":
            _, X, W = t
            add(X, pl.BlockSpec((BLK, X.shape[1]), lambda i: (i, 0)))
            add(W, pl.BlockSpec(W.shape, lambda i: (0, 0)))
        elif kind == "parts":
            _, P, dv, W = t
            add(P, pl.BlockSpec((2, BLK, H), lambda i: (0, i, 0)))
            if dv is not None:
                add(dv, pl.BlockSpec((BLK, 1), lambda i: (i, 0)))
            add(W, pl.BlockSpec(W.shape, lambda i: (0, 0)))
        elif kind == "onehot":
            _, ids3, T, sc = t
            add(ids3, pl.BlockSpec((1, 1, BLK), lambda i: (i, 0, 0)))
            add(T, pl.BlockSpec(T.shape, lambda i: (0, 0)))
            if sc is not None:
                add(sc, pl.BlockSpec((BLK, 1), lambda i: (i, 0)))
        elif kind == "gparts":
            _, P, sc, S = t
            add(P, pl.BlockSpec((2, BLK, out_dim), lambda i: (0, i, 0)))
            add(sc, pl.BlockSpec((BLK, 1), lambda i: (i, 0)))
            add(S, pl.BlockSpec((1, 1), lambda i: (0, 0)))
    if bias is not None:
        add(bias.reshape(1, out_dim), pl.BlockSpec((1, out_dim), lambda i: (0, 0)))
    if residual is not None:
        add(residual, pl.BlockSpec((BLK, out_dim), lambda i: (i, 0)))

    def body(*refs):
        refs = list(refs)
        out_ref = refs.pop()
        acc = jnp.zeros((BLK, out_dim), jnp.float32)
        for t in terms:
            kind = t[0]
            if kind == "mm":
                xr = refs.pop(0)[...]
                wr = refs.pop(0)[...]
                acc += _dot(xr, wr)
            elif kind == "parts":
                pr = refs.pop(0)[...]
                p = pr[0] + pr[1]
                if t[2] is not None:
                    p = p / refs.pop(0)[...]
                acc += _dot(p, refs.pop(0)[...])
            elif kind == "onehot":
                ids = refs.pop(0)[0, 0, :]
                T = refs.pop(0)[...]
                C = T.shape[0]
                oh = (ids[:, None] ==
                      lax.broadcasted_iota(jnp.int32, (BLK, C), 1)).astype(jnp.float32)
                g = _dot(oh, T)
                if t[3] is not None:
                    g = g * refs.pop(0)[...]
                acc += g
            elif kind == "gparts":
                pr = refs.pop(0)[...]
                scr = refs.pop(0)[...]
                Sr = refs.pop(0)[0, 0]
                acc += (pr[0] + pr[1]) * (scr / Sr)
        if bias is not None:
            acc += refs.pop(0)[...]
        if act == "lrelu":
            acc = jnp.maximum(acc, 0.01 * acc)
        elif act == "sigmoid":
            acc = jax.nn.sigmoid(acc)
        if residual is not None:
            acc += refs.pop(0)[...]
        out_ref[...] = acc

    return pl.pallas_call(
        body,
        grid=(nb,),
        in_specs=specs,
        out_specs=pl.BlockSpec((BLK, out_dim), lambda i: (i, 0)),
        out_shape=jax.ShapeDtypeStruct((n, out_dim), jnp.float32),
    )(*args)


def _cluster_sum(x, ids3):
    def body(ids_ref, x_ref, out_ref):
        @pl.when(pl.program_id(0) == 0)
        def _():
            out_ref[...] = jnp.zeros_like(out_ref)
        ids = ids_ref[0, 0, :]
        oh = (ids[:, None] ==
              lax.broadcasted_iota(jnp.int32, (BLK, NC_PAD), 1)).astype(jnp.float32)
        out_ref[...] += _dott(oh, x_ref[...])

    return pl.pallas_call(
        body,
        grid=(N_PAD // BLK,),
        in_specs=[pl.BlockSpec((1, 1, BLK), lambda i: (i, 0, 0)),
                  pl.BlockSpec((BLK, H), lambda i: (i, 0))],
        out_specs=pl.BlockSpec((NC_PAD, H), lambda i: (0, 0)),
        out_shape=jax.ShapeDtypeStruct((NC_PAD, H), jnp.float32),
    )(ids3, x)


def _cluster_counts(ids3):
    def body(ids_ref, out_ref):
        @pl.when(pl.program_id(0) == 0)
        def _():
            out_ref[...] = jnp.zeros_like(out_ref)
        ids = ids_ref[0, 0, :]
        oh = (ids[:, None] ==
              lax.broadcasted_iota(jnp.int32, (BLK, NC_PAD), 1)).astype(jnp.float32)
        out_ref[...] += _dott(oh, jnp.ones((BLK, 1), jnp.float32))

    return pl.pallas_call(
        body,
        grid=(N_PAD // BLK,),
        in_specs=[pl.BlockSpec((1, 1, BLK), lambda i: (i, 0, 0))],
        out_specs=pl.BlockSpec((NC_PAD, 1), lambda i: (0, 0)),
        out_shape=jax.ShapeDtypeStruct((NC_PAD, 1), jnp.float32),
    )(ids3)


def _cm_proj(csum, ccnt, U3):
    def body(cs_ref, cc_ref, u_ref, out_ref):
        cm = cs_ref[...] / jnp.maximum(cc_ref[...], 1.0)
        out_ref[...] = _dot(cm, u_ref[...])

    return pl.pallas_call(
        body,
        out_shape=jax.ShapeDtypeStruct((NC_PAD, H), jnp.float32),
    )(csum, ccnt, U3)


def _deg_reduce(parts):
    def body(p_ref, out_ref):
        p = p_ref[...]
        out_ref[...] = jnp.maximum(p[0, :, :1] + p[1, :, :1], 1.0)

    return pl.pallas_call(
        body,
        grid=(N_PAD // BLK,),
        in_specs=[pl.BlockSpec((2, BLK, 16), lambda i: (0, i, 0))],
        out_specs=pl.BlockSpec((BLK, 1), lambda i: (i, 0)),
        out_shape=jax.ShapeDtypeStruct((N_PAD, 1), jnp.float32),
    )(parts)


_ZROWS = EC_PAD // 128
_ZBLK = 16


def _ptr_phase_a(zp, u2d):
    nb = _ZROWS // _ZBLK

    def body(zp_ref, u_ref, z_ref, m_ref):
        i = pl.program_id(0)
        s = jnp.sum(zp_ref[...], axis=1).reshape(_ZBLK, 128)
        u = u_ref[...]
        g = -jnp.log(-jnp.log(u))
        ids = (i * (_ZBLK * 128) +
               lax.broadcasted_iota(jnp.int32, (_ZBLK, 128), 0) * 128 +
               lax.broadcasted_iota(jnp.int32, (_ZBLK, 128), 1))
        z = jnp.where(ids < EC_REAL, s + g, -1e30)
        z_ref[...] = z

        @pl.when(i == 0)
        def _():
            m_ref[...] = jnp.full((1, 1), -1e30, jnp.float32)
        m_ref[...] = jnp.maximum(m_ref[...], jnp.max(z))

    return pl.pallas_call(
        body,
        grid=(nb,),
        in_specs=[pl.BlockSpec((_ZBLK * 128, 16), lambda i: (i, 0)),
                  pl.BlockSpec((_ZBLK, 128), lambda i: (i, 0))],
        out_specs=[pl.BlockSpec((_ZBLK, 128), lambda i: (i, 0)),
                   pl.BlockSpec((1, 1), lambda i: (0, 0))],
        out_shape=[jax.ShapeDtypeStruct((_ZROWS, 128), jnp.float32),
                   jax.ShapeDtypeStruct((1, 1), jnp.float32)],
    )(zp, u2d)


def _ptr_phase_b(z2d, m):
    nb = _ZROWS // _ZBLK

    def body(z_ref, m_ref, y_ref, s_ref):
        i = pl.program_id(0)
        y = jnp.exp(z_ref[...] - m_ref[0, 0])
        y_ref[...] = y

        @pl.when(i == 0)
        def _():
            s_ref[...] = jnp.zeros((1, 1), jnp.float32)
        s_ref[...] += jnp.sum(y)

    return pl.pallas_call(
        body,
        grid=(nb,),
        in_specs=[pl.BlockSpec((_ZBLK, 128), lambda i: (i, 0)),
                  pl.BlockSpec((1, 1), lambda i: (0, 0))],
        out_specs=[pl.BlockSpec((_ZBLK, 128), lambda i: (i, 0)),
                   pl.BlockSpec((1, 1), lambda i: (0, 0))],
        out_shape=[jax.ShapeDtypeStruct((_ZROWS, 128), jnp.float32),
                   jax.ShapeDtypeStruct((1, 1), jnp.float32)],
    )(z2d, m)


_MESH = plsc.VectorSubcoreMesh(core_axis_name="c", subcore_axis_name="s")
_ZR = N_ACC // 16
_NTAIL = N_PAD - N_ACC


def _sc_worker_ids():
    c = lax.axis_index("c")
    s = lax.axis_index("s")
    return c, s, s * 2 + c


def _sc_edge_aggr(A, B, src3, dst3, zeros_big, nq):
    @functools.partial(
        pl.kernel,
        out_type=jax.ShapeDtypeStruct((2, N_PAD, H), jnp.float32),
        mesh=_MESH,
        scratch_types=[
            pltpu.VMEM((8, CHW), jnp.int32),
            pltpu.VMEM((8, CHW), jnp.int32),
            pltpu.VMEM((2, CHW, H), jnp.float32),
            pltpu.VMEM((2, CHW, H), jnp.float32),
            pltpu.VMEM_SHARED((N_ACC, H), jnp.float32),
            pltpu.SemaphoreType.DMA,
            pltpu.SemaphoreType.DMA,
            pltpu.SemaphoreType.DMA,
            pltpu.SemaphoreType.DMA,
            pltpu.SemaphoreType.DMA,
            pltpu.SemaphoreType.DMA,
        ],
    )
    def k(A_hbm, B_hbm, src_hbm, dst_hbm, z_hbm, out_hbm,
          sidx, didx, rowsA, rowsB, acc,
          semA0, semA1, semB0, semB1, semS0, semS1):
        c, s, wid = _sc_worker_ids()
        base = wid * nq
        semA = (semA0, semA1)
        semB = (semB0, semB1)
        semS = (semS0, semS1)
        pltpu.sync_copy(z_hbm.at[pl.ds(s * _ZR, _ZR)], acc.at[pl.ds(s * _ZR, _ZR)])
        plsc.subcore_barrier()

        def gather(cc, p):
            ga = pltpu.async_copy(A_hbm.at[didx.at[cc]], rowsA.at[p], semA[p])
            gb = pltpu.async_copy(B_hbm.at[sidx.at[cc]], rowsB.at[p], semB[p])
            return ga, gb

        def compute(p):
            def row(r, carry2):
                for kk in range(8):
                    a = (rowsA[p, r, pl.ds(kk * 16, 16)] +
                         rowsB[p, r, pl.ds(kk * 16, 16)])
                    rowsA[p, r, pl.ds(kk * 16, 16)] = jnp.maximum(a, 0.01 * a)
                return carry2
            lax.fori_loop(0, CHW, row, 0)

        def superchunk(q, carry):
            pltpu.sync_copy(src_hbm.at[base + q], sidx)
            pltpu.sync_copy(dst_hbm.at[base + q], didx)
            g = [None, None]
            sc_pend = [None, None]
            g[0] = gather(0, 0)
            for cc in range(8):
                p = cc & 1
                g[p][0].wait()
                g[p][1].wait()
                if cc < 7:
                    if sc_pend[1 - p] is not None:
                        sc_pend[1 - p].wait()
                        sc_pend[1 - p] = None
                    g[1 - p] = gather(cc + 1, 1 - p)
                compute(p)
                sc_pend[p] = pltpu.async_copy(
                    rowsA.at[p], acc.at[didx.at[cc]], semS[p], add=True)
            sc_pend[0].wait()
            sc_pend[1].wait()
            return carry

        lax.fori_loop(0, nq, superchunk, 0)
        plsc.subcore_barrier()
        pltpu.sync_copy(acc.at[pl.ds(s * _ZR, _ZR)],
                        out_hbm.at[c, pl.ds(s * _ZR, _ZR)])

        @pl.when(s == 0)
        def _():
            pltpu.sync_copy(z_hbm.at[pl.ds(0, _NTAIL)],
                            out_hbm.at[c, pl.ds(N_ACC, _NTAIL)])

    return k(A, B, src3, dst3, zeros_big)


def _sc_degree(dst3, ones16, zeros16, nq):
    @functools.partial(
        pl.kernel,
        out_type=jax.ShapeDtypeStruct((2, N_PAD, 16), jnp.float32),
        mesh=_MESH,
        scratch_types=[
            pltpu.VMEM((8, CHW), jnp.int32),
            pltpu.VMEM((CHW, 16), jnp.float32),
            pltpu.VMEM_SHARED((N_ACC, 16), jnp.float32),
            pltpu.SemaphoreType.DMA,
        ],
    )
    def k(dst_hbm, ones_hbm, z_hbm, out_hbm, didx, ones_v, acc, semS):
        c, s, wid = _sc_worker_ids()
        base = wid * nq
        pltpu.sync_copy(ones_hbm, ones_v)
        pltpu.sync_copy(z_hbm.at[pl.ds(s * _ZR, _ZR), pl.ds(0, 16)],
                        acc.at[pl.ds(s * _ZR, _ZR)])
        plsc.subcore_barrier()

        def superchunk(q, carry):
            pltpu.sync_copy(dst_hbm.at[base + q], didx)
            pend = []
            for cc in range(8):
                pend.append(pltpu.async_copy(
                    ones_v, acc.at[didx.at[cc]], semS, add=True))
            for d in pend:
                d.wait()
            return carry

        lax.fori_loop(0, nq, superchunk, 0)
        plsc.subcore_barrier()
        pltpu.sync_copy(acc.at[pl.ds(s * _ZR, _ZR)],
                        out_hbm.at[c, pl.ds(s * _ZR, _ZR)])

        @pl.when(s == 0)
        def _():
            pltpu.sync_copy(z_hbm.at[pl.ds(0, _NTAIL), pl.ds(0, 16)],
                            out_hbm.at[c, pl.ds(N_ACC, _NTAIL)])

    return k(dst3, ones16, zeros16)


def _sc_ptr_score(Xp, Vv, ce0_3d, ce1_3d, theta816, nq):
    @functools.partial(
        pl.kernel,
        out_type=jax.ShapeDtypeStruct((EC_PAD // CHW, CHW, 16), jnp.float32),
        mesh=_MESH,
        scratch_types=[
            pltpu.VMEM((8, CHW), jnp.int32),
            pltpu.VMEM((8, CHW), jnp.int32),
            pltpu.VMEM((2, CHW, H), jnp.float32),
            pltpu.VMEM((2, CHW, H), jnp.float32),
            pltpu.VMEM((2, CHW, 16), jnp.float32),
            pltpu.VMEM((8, 16), jnp.float32),
            pltpu.SemaphoreType.DMA,
            pltpu.SemaphoreType.DMA,
            pltpu.SemaphoreType.DMA,
            pltpu.SemaphoreType.DMA,
            pltpu.SemaphoreType.DMA,
            pltpu.SemaphoreType.DMA,
        ],
    )
    def k(Xp_hbm, Vv_hbm, i0_hbm, i1_hbm, th_hbm, out_hbm,
          i0, i1, rowsX, rowsV, zbuf, thv,
          semA0, semA1, semB0, semB1, semS0, semS1):
        c, s, wid = _sc_worker_ids()
        base = wid * nq
        semA = (semA0, semA1)
        semB = (semB0, semB1)
        semS = (semS0, semS1)
        pltpu.sync_copy(th_hbm, thv)

        def gather(cc, p):
            ga = pltpu.async_copy(Xp_hbm.at[i0.at[cc]], rowsX.at[p], semA[p])
            gb = pltpu.async_copy(Vv_hbm.at[i1.at[cc]], rowsV.at[p], semB[p])
            return ga, gb

        def compute(p):
            th = [thv[kk, :] for kk in range(8)]

            def row(r, carry2):
                acc = jnp.zeros((16,), jnp.float32)
                for kk in range(8):
                    zz = (rowsX[p, r, pl.ds(kk * 16, 16)] +
                          rowsV[p, r, pl.ds(kk * 16, 16)])
                    t = 1.0 - 2.0 / (jnp.exp(2.0 * zz) + 1.0)
                    acc = acc + th[kk] * t
                zbuf[p, r, :] = acc
                return carry2
            lax.fori_loop(0, CHW, row, 0)

        def superchunk(q, carry):
            pltpu.sync_copy(i0_hbm.at[base + q], i0)
            pltpu.sync_copy(i1_hbm.at[base + q], i1)
            g = [None, None]
            st_pend = [None, None]
            g[0] = gather(0, 0)
            for cc in range(8):
                p = cc & 1
                g[p][0].wait()
                g[p][1].wait()
                if cc < 7:
                    g[1 - p] = gather(cc + 1, 1 - p)
                if st_pend[p] is not None:
                    st_pend[p].wait()
                    st_pend[p] = None
                compute(p)
                st_pend[p] = pltpu.async_copy(
                    zbuf.at[p], out_hbm.at[(base + q) * 8 + cc], semS[p])
            st_pend[0].wait()
            st_pend[1].wait()
            return carry

        lax.fori_loop(0, nq, superchunk, 0)

    return k(Xp, Vv, ce0_3d, ce1_3d, theta816)


def _sc_ptr_scatter(x, ce0_3d, ce1_3d, y3d, zeros_big, nq):
    @functools.partial(
        pl.kernel,
        out_type=jax.ShapeDtypeStruct((2, N_PAD, H), jnp.float32),
        mesh=_MESH,
        scratch_types=[
            pltpu.VMEM((8, CHW), jnp.int32),
            pltpu.VMEM((8, CHW), jnp.int32),
            pltpu.VMEM((2, CHW, H), jnp.float32),
            pltpu.VMEM((8, CHW), jnp.float32),
            pltpu.VMEM_SHARED((N_ACC, H), jnp.float32),
            pltpu.SemaphoreType.DMA,
            pltpu.SemaphoreType.DMA,
            pltpu.SemaphoreType.DMA,
            pltpu.SemaphoreType.DMA,
        ],
    )
    def k(x_hbm, i0_hbm, i1_hbm, y_hbm, z_hbm, out_hbm,
          i0, i1, rowsX, ybuf, acc, semA0, semA1, semS0, semS1):
        c, s, wid = _sc_worker_ids()
        base = wid * nq
        semA = (semA0, semA1)
        semS = (semS0, semS1)
        pltpu.sync_copy(z_hbm.at[pl.ds(s * _ZR, _ZR)], acc.at[pl.ds(s * _ZR, _ZR)])
        plsc.subcore_barrier()

        def compute(p, cc):
            def grp(gg, carry2):
                yvec = ybuf[cc, pl.ds(gg * 16, 16)]
                for lane in range(16):
                    r = gg * 16 + lane
                    yv = yvec[lane]
                    for kk in range(8):
                        rowsX[p, r, pl.ds(kk * 16, 16)] = (
                            rowsX[p, r, pl.ds(kk * 16, 16)] * yv)
                return carry2
            lax.fori_loop(0, CHW // 16, grp, 0)

        def superchunk(q, carry):
            pltpu.sync_copy(i0_hbm.at[base + q], i0)
            pltpu.sync_copy(i1_hbm.at[base + q], i1)
            pltpu.sync_copy(y_hbm.at[base + q], ybuf)
            g = [None, None]
            sc_pend = [None, None]
            g[0] = pltpu.async_copy(x_hbm.at[i0.at[0]], rowsX.at[0], semA[0])
            for cc in range(8):
                p = cc & 1
                g[p].wait()
                if cc < 7:
                    if sc_pend[1 - p] is not None:
                        sc_pend[1 - p].wait()
                        sc_pend[1 - p] = None
                    g[1 - p] = pltpu.async_copy(
                        x_hbm.at[i0.at[cc + 1]], rowsX.at[1 - p], semA[1 - p])
                compute(p, cc)
                sc_pend[p] = pltpu.async_copy(
                    rowsX.at[p], acc.at[i1.at[cc]], semS[p], add=True)
            sc_pend[0].wait()
            sc_pend[1].wait()
            return carry

        lax.fori_loop(0, nq, superchunk, 0)
        plsc.subcore_barrier()
        pltpu.sync_copy(acc.at[pl.ds(s * _ZR, _ZR)],
                        out_hbm.at[c, pl.ds(s * _ZR, _ZR)])

        @pl.when(s == 0)
        def _():
            pltpu.sync_copy(z_hbm.at[pl.ds(0, _NTAIL)],
                            out_hbm.at[c, pl.ds(N_ACC, _NTAIL)])

    return k(x, ce0_3d, ce1_3d, y3d, zeros_big)


def _pad_rows(a, n):
    return jnp.pad(a, ((0, n - a.shape[0]), (0, 0)))


def _prep_edges(ei, epad):
    e = ei.shape[1]
    p = jnp.pad(ei, ((0, 0), (0, epad - e)), constant_values=DUMMY)
    return (p[0].reshape(epad // SCH, 8, CHW), p[1].reshape(epad // SCH, 8, CHW))


def _pe_host():
    pos = np.arange(100, dtype=np.float32)[:, None]
    i2 = np.arange(0, H, 2, dtype=np.float32)
    ang = pos / (10000.0 ** (i2 / H))
    t = np.zeros((128, H), dtype=np.float32)
    t[:100, 0::2] = np.sin(ang)
    t[:100, 1::2] = np.cos(ang)
    return jnp.asarray(t)


def kernel(local_x, local_edge_index, node_cluster, node_ratio, voxel_x,
           voxel_edge_index, voxel_level, cross_edge_index, program_noise,
           voxel_noise, params):
    nq_p = EP_PAD // NW // SCH
    nq_v = EV_PAD // NW // SCH
    src_p, dst_p = _prep_edges(local_edge_index, EP_PAD)
    src_v, dst_v = _prep_edges(voxel_edge_index, EV_PAD)
    ce0, ce1 = _prep_edges(cross_edge_index, EC_PAD)

    ids3 = jnp.pad(node_cluster, (0, N_PAD - N_REAL),
                   constant_values=NC_PAD - 1).reshape(N_PAD // BLK, 1, BLK)
    lvl3 = jnp.pad(voxel_level, (0, N_PAD - N_REAL)).reshape(N_PAD // BLK, 1, BLK)
    ratio = _pad_rows(jnp.sum(node_ratio, axis=1)[:, None], N_PAD)

    zeros_big = jnp.zeros((N_PAD, H), jnp.float32)
    zeros16 = jnp.zeros((N_PAD, 16), jnp.float32)
    ones16 = jnp.ones((CHW, 16), jnp.float32)
    pe_pad = _pe_host()

    u2d = {}
    for li in (1, 3):
        u = jax.random.uniform(jax.random.fold_in(jax.random.key(42), li),
                               (EC_REAL,), minval=1e-9, maxval=1.0,
                               dtype=jnp.float32)
        u2d[li] = jnp.pad(u, (0, EC_PAD - EC_REAL),
                          constant_values=0.5).reshape(_ZROWS, 128)

    degp = _sc_degree(dst_p, ones16, zeros16, nq_p)
    deg = _deg_reduce(degp)
    ccnt = _cluster_counts(ids3)

    pW, pb = params["p_enc"]["W"], params["p_enc"]["b"]
    x = _fused_rows(N_PAD,
                    [("mm", _pad_rows(local_x, N_PAD), pW[:128]),
                     ("mm", _pad_rows(program_noise, N_PAD), pW[128:])],
                    bias=pb, act="lrelu")
    pos = _fused_rows(N_PAD, [("onehot", lvl3, pe_pad, None)])
    vW, vb = params["v_enc"]["W"], params["v_enc"]["b"]
    v = _fused_rows(N_PAD,
                    [("mm", _pad_rows(voxel_x, N_PAD), vW[:128]),
                     ("mm", _pad_rows(voxel_noise, N_PAD), vW[128:])],
                    bias=vb, act="lrelu", residual=pos)

    for l in range(P_STEPS):
        W, b = params["p_msg"][l]["W"], params["p_msg"][l]["b"]
        A = _fused_rows(N_PAD, [("mm", x, W[:128])], bias=b)
        B = _fused_rows(N_PAD, [("mm", x, W[128:])])
        csum = _cluster_sum(x, ids3)
        U, ub = params["p_upd"][l]["W"], params["p_upd"][l]["b"]
        CU = _cm_proj(csum, ccnt, U[256:])
        parts = _sc_edge_aggr(A, B, src_p, dst_p, zeros_big, nq_p)
        x = _fused_rows(N_PAD,
                        [("mm", x, U[:128]),
                         ("parts", parts, deg, U[128:256]),
                         ("onehot", ids3, CU, ratio)],
                        bias=ub, act="lrelu", residual=x)

    ptr = params["ptr"]
    Xp = _fused_rows(N_PAD, [("mm", x, ptr["Wp"]["W"])], bias=ptr["Wp"]["b"])
    theta816 = ptr["theta"][:, 0].reshape(8, 16)

    for li in range(V_STEPS):
        W, b = params["v_msg"][li]["W"], params["v_msg"][li]["b"]
        A = _fused_rows(N_PAD, [("mm", v, W[:128]), ("mm", pos, W[256:])], bias=b)
        B = _fused_rows(N_PAD, [("mm", v, W[128:256]), ("mm", pos, -W[256:])])
        parts = _sc_edge_aggr(A, B, src_v, dst_v, zeros_big, nq_v)
        U, ub = params["v_upd"][li]["W"], params["v_upd"][li]["b"]
        v = _fused_rows(N_PAD,
                        [("mm", v, U[:128]), ("parts", parts, None, U[128:])],
                        bias=ub, act="lrelu", residual=v)
        if (li + 1) % 2 == 0:
            Vv = _fused_rows(N_PAD, [("mm", v, ptr["Wv"]["W"])],
                             bias=ptr["Wv"]["b"])
            h = _fused_rows(N_PAD, [("mm", v, ptr["m1"]["W"])],
                            bias=ptr["m1"]["b"], act="lrelu")
            mask = _fused_rows(N_PAD, [("mm", h, ptr["m2"]["W"])],
                               bias=ptr["m2"]["b"], act="sigmoid", out_dim=1)
            zp = _sc_ptr_score(Xp, Vv, ce0, ce1, theta816, nq_v)
            z2d, m = _ptr_phase_a(zp.reshape(EC_PAD, 16), u2d[li])
            y2d, S = _ptr_phase_b(z2d, m)
            y3d = y2d.reshape(EC_PAD // SCH, 8, CHW)
            parts = _sc_ptr_scatter(x, ce0, ce1, y3d, zeros_big, nq_v)
            v = _fused_rows(N_PAD, [("gparts", parts, mask, S)], residual=v)

    return v[:N_REAL]


# speedup vs baseline: 1.5245x; 1.3635x over previous
import functools

import jax
import jax.numpy as jnp
import numpy as np
from jax import lax
from jax.experimental import pallas as pl
from jax.experimental.pallas import tpu as pltpu
from jax.experimental.pallas import tpu_sc as plsc

H = 128
N_REAL = 10000
N_PAD = 10240
N_ACC = 10112
NC_PAD = 512
DUMMY = N_ACC - 1
CHW = 80
SCH = 8 * CHW
EP_PAD = 163840
EV_PAD = 327680
EC_PAD = 327680
EC_REAL = 320000
NW = 32
BLK = 1024
P_STEPS = 3
V_STEPS = 4

_HIGH = jax.lax.Precision.HIGHEST


def _dot(a, b):
    return lax.dot_general(a, b, (((1,), (0,)), ((), ())),
                           preferred_element_type=jnp.float32,
                           precision=_HIGH)


def _dott(a, b):
    return lax.dot_general(a, b, (((0,), (0,)), ((), ())),
                           preferred_element_type=jnp.float32,
                           precision=_HIGH)


def _pack_bf16_pairs(acc):
    bits = lax.bitcast_convert_type(acc, jnp.int32)
    u = bits + jnp.int32(0x7FFF) + ((bits >> 16) & 1)
    h = lax.shift_right_logical(u, 16)
    return (h[:, 64:] << 16) | (h[:, :64] & jnp.int32(0xFFFF))


def _fused_rows(n, terms, bias=None, act="none", residual=None, out_dim=H,
                pack=False):
    nb = n // BLK
    args, specs = [], []

    def add(a, spec):
        args.append(a)
        specs.append(spec)

    for t in terms:
        kind = t[0]
        if kind == "mm":
            _, X, W = t
            add(X, pl.BlockSpec((BLK, X.shape[1]), lambda i: (i, 0)))
            add(W, pl.BlockSpec(W.shape, lambda i: (0, 0)))
        elif kind == "parts":
            _, P, dv, W = t
            add(P, pl.BlockSpec((2, BLK, H), lambda i: (0, i, 0)))
            if dv is not None:
                add(dv, pl.BlockSpec((BLK, 1), lambda i: (i, 0)))
            add(W, pl.BlockSpec(W.shape, lambda i: (0, 0)))
        elif kind == "onehot":
            _, ids3, T, sc = t
            add(ids3, pl.BlockSpec((1, 1, BLK), lambda i: (i, 0, 0)))
            add(T, pl.BlockSpec(T.shape, lambda i: (0, 0)))
            if sc is not None:
                add(sc, pl.BlockSpec((BLK, 1), lambda i: (i, 0)))
        elif kind == "gparts":
            _, P, sc, S = t
            add(P, pl.BlockSpec((2, BLK, out_dim), lambda i: (0, i, 0)))
            add(sc, pl.BlockSpec((BLK, 1), lambda i: (i, 0)))
            add(S, pl.BlockSpec((1, 1), lambda i: (0, 0)))
    if bias is not None:
        add(bias.reshape(1, out_dim), pl.BlockSpec((1, out_dim), lambda i: (0, 0)))
    if residual is not None:
        add(residual, pl.BlockSpec((BLK, out_dim), lambda i: (i, 0)))

    def body(*refs):
        refs = list(refs)
        out_ref = refs.pop()
        acc = jnp.zeros((BLK, out_dim), jnp.float32)
        for t in terms:
            kind = t[0]
            if kind == "mm":
                xr = refs.pop(0)[...]
                wr = refs.pop(0)[...]
                acc += _dot(xr, wr)
            elif kind == "parts":
                pr = refs.pop(0)[...]
                p = pr[0] + pr[1]
                if t[2] is not None:
                    p = p / refs.pop(0)[...]
                acc += _dot(p, refs.pop(0)[...])
            elif kind == "onehot":
                ids = refs.pop(0)[0, 0, :]
                T = refs.pop(0)[...]
                C = T.shape[0]
                oh = (ids[:, None] ==
                      lax.broadcasted_iota(jnp.int32, (BLK, C), 1)).astype(jnp.float32)
                g = _dot(oh, T)
                if t[3] is not None:
                    g = g * refs.pop(0)[...]
                acc += g
            elif kind == "gparts":
                pr = refs.pop(0)[...]
                scr = refs.pop(0)[...]
                Sr = refs.pop(0)[0, 0]
                acc += (pr[0] + pr[1]) * (scr / Sr)
        if bias is not None:
            acc += refs.pop(0)[...]
        if act == "lrelu":
            acc = jnp.maximum(acc, 0.01 * acc)
        elif act == "sigmoid":
            acc = jax.nn.sigmoid(acc)
        if residual is not None:
            acc += refs.pop(0)[...]
        if pack:
            out_ref[...] = _pack_bf16_pairs(acc)
        else:
            out_ref[...] = acc

    if pack:
        ospec = pl.BlockSpec((BLK, out_dim // 2), lambda i: (i, 0))
        oshape = jax.ShapeDtypeStruct((n, out_dim // 2), jnp.int32)
    else:
        ospec = pl.BlockSpec((BLK, out_dim), lambda i: (i, 0))
        oshape = jax.ShapeDtypeStruct((n, out_dim), jnp.float32)
    return pl.pallas_call(
        body,
        grid=(nb,),
        in_specs=specs,
        out_specs=ospec,
        out_shape=oshape,
    )(*args)


def _pack_rows(x):
    def body(x_ref, out_ref):
        out_ref[...] = _pack_bf16_pairs(x_ref[...])

    return pl.pallas_call(
        body,
        grid=(N_PAD // BLK,),
        in_specs=[pl.BlockSpec((BLK, H), lambda i: (i, 0))],
        out_specs=pl.BlockSpec((BLK, H // 2), lambda i: (i, 0)),
        out_shape=jax.ShapeDtypeStruct((N_PAD, H // 2), jnp.int32),
    )(x)


def _cluster_sum(x, ids3):
    def body(ids_ref, x_ref, out_ref):
        @pl.when(pl.program_id(0) == 0)
        def _():
            out_ref[...] = jnp.zeros_like(out_ref)
        ids = ids_ref[0, 0, :]
        oh = (ids[:, None] ==
              lax.broadcasted_iota(jnp.int32, (BLK, NC_PAD), 1)).astype(jnp.float32)
        out_ref[...] += _dott(oh, x_ref[...])

    return pl.pallas_call(
        body,
        grid=(N_PAD // BLK,),
        in_specs=[pl.BlockSpec((1, 1, BLK), lambda i: (i, 0, 0)),
                  pl.BlockSpec((BLK, H), lambda i: (i, 0))],
        out_specs=pl.BlockSpec((NC_PAD, H), lambda i: (0, 0)),
        out_shape=jax.ShapeDtypeStruct((NC_PAD, H), jnp.float32),
    )(ids3, x)


def _cluster_counts(ids3):
    def body(ids_ref, out_ref):
        @pl.when(pl.program_id(0) == 0)
        def _():
            out_ref[...] = jnp.zeros_like(out_ref)
        ids = ids_ref[0, 0, :]
        oh = (ids[:, None] ==
              lax.broadcasted_iota(jnp.int32, (BLK, NC_PAD), 1)).astype(jnp.float32)
        out_ref[...] += _dott(oh, jnp.ones((BLK, 1), jnp.float32))

    return pl.pallas_call(
        body,
        grid=(N_PAD // BLK,),
        in_specs=[pl.BlockSpec((1, 1, BLK), lambda i: (i, 0, 0))],
        out_specs=pl.BlockSpec((NC_PAD, 1), lambda i: (0, 0)),
        out_shape=jax.ShapeDtypeStruct((NC_PAD, 1), jnp.float32),
    )(ids3)


def _cm_proj(csum, ccnt, U3):
    def body(cs_ref, cc_ref, u_ref, out_ref):
        cm = cs_ref[...] / jnp.maximum(cc_ref[...], 1.0)
        out_ref[...] = _dot(cm, u_ref[...])

    return pl.pallas_call(
        body,
        out_shape=jax.ShapeDtypeStruct((NC_PAD, H), jnp.float32),
    )(csum, ccnt, U3)


def _deg_reduce(parts):
    def body(p_ref, out_ref):
        p = p_ref[...]
        out_ref[...] = jnp.maximum(p[0, :, :1] + p[1, :, :1], 1.0)

    return pl.pallas_call(
        body,
        grid=(N_PAD // BLK,),
        in_specs=[pl.BlockSpec((2, BLK, 16), lambda i: (0, i, 0))],
        out_specs=pl.BlockSpec((BLK, 1), lambda i: (i, 0)),
        out_shape=jax.ShapeDtypeStruct((N_PAD, 1), jnp.float32),
    )(parts)


_ZROWS = EC_PAD // 128
_ZBLK = 16


def _ptr_phase_a(zp, u2d):
    nb = _ZROWS // _ZBLK

    def body(zp_ref, u_ref, z_ref, m_ref):
        i = pl.program_id(0)
        s = jnp.sum(zp_ref[...], axis=1).reshape(_ZBLK, 128)
        u = u_ref[...]
        g = -jnp.log(-jnp.log(u))
        ids = (i * (_ZBLK * 128) +
               lax.broadcasted_iota(jnp.int32, (_ZBLK, 128), 0) * 128 +
               lax.broadcasted_iota(jnp.int32, (_ZBLK, 128), 1))
        z = jnp.where(ids < EC_REAL, s + g, -1e30)
        z_ref[...] = z

        @pl.when(i == 0)
        def _():
            m_ref[...] = jnp.full((1, 1), -1e30, jnp.float32)
        m_ref[...] = jnp.maximum(m_ref[...], jnp.max(z))

    return pl.pallas_call(
        body,
        grid=(nb,),
        in_specs=[pl.BlockSpec((_ZBLK * 128, 16), lambda i: (i, 0)),
                  pl.BlockSpec((_ZBLK, 128), lambda i: (i, 0))],
        out_specs=[pl.BlockSpec((_ZBLK, 128), lambda i: (i, 0)),
                   pl.BlockSpec((1, 1), lambda i: (0, 0))],
        out_shape=[jax.ShapeDtypeStruct((_ZROWS, 128), jnp.float32),
                   jax.ShapeDtypeStruct((1, 1), jnp.float32)],
    )(zp, u2d)


def _ptr_phase_b(z2d, m):
    nb = _ZROWS // _ZBLK

    def body(z_ref, m_ref, y_ref, s_ref):
        i = pl.program_id(0)
        y = jnp.exp(z_ref[...] - m_ref[0, 0])
        y_ref[...] = y

        @pl.when(i == 0)
        def _():
            s_ref[...] = jnp.zeros((1, 1), jnp.float32)
        s_ref[...] += jnp.sum(y)

    return pl.pallas_call(
        body,
        grid=(nb,),
        in_specs=[pl.BlockSpec((_ZBLK, 128), lambda i: (i, 0)),
                  pl.BlockSpec((1, 1), lambda i: (0, 0))],
        out_specs=[pl.BlockSpec((_ZBLK, 128), lambda i: (i, 0)),
                   pl.BlockSpec((1, 1), lambda i: (0, 0))],
        out_shape=[jax.ShapeDtypeStruct((_ZROWS, 128), jnp.float32),
                   jax.ShapeDtypeStruct((1, 1), jnp.float32)],
    )(z2d, m)


_MESH = plsc.VectorSubcoreMesh(core_axis_name="c", subcore_axis_name="s")
_ZR = N_ACC // 16
_NTAIL = N_PAD - N_ACC


def _sc_worker_ids():
    c = lax.axis_index("c")
    s = lax.axis_index("s")
    return c, s, s * 2 + c


def _sc_edge_aggr(A, B, src3, dst3, zeros_big, nq):
    @functools.partial(
        pl.kernel,
        out_type=jax.ShapeDtypeStruct((2, N_PAD, H), jnp.float32),
        mesh=_MESH,
        compiler_params=pltpu.CompilerParams(use_tc_tiling_on_sc=False),
        scratch_types=[
            pltpu.VMEM((8, CHW), jnp.int32),
            pltpu.VMEM((8, CHW), jnp.int32),
            pltpu.VMEM((2, CHW, H // 2), jnp.int32),
            pltpu.VMEM((2, CHW, H // 2), jnp.int32),
            pltpu.VMEM((2, CHW, H), jnp.float32),
            pltpu.VMEM_SHARED((N_ACC, H), jnp.float32),
            pltpu.SemaphoreType.DMA,
            pltpu.SemaphoreType.DMA,
            pltpu.SemaphoreType.DMA,
            pltpu.SemaphoreType.DMA,
            pltpu.SemaphoreType.DMA,
            pltpu.SemaphoreType.DMA,
        ],
    )
    def k(A_hbm, B_hbm, src_hbm, dst_hbm, z_hbm, out_hbm,
          sidx, didx, rowsA, rowsB, msg, acc,
          semA0, semA1, semB0, semB1, semS0, semS1):
        c, s, wid = _sc_worker_ids()
        base = wid * nq
        semA = (semA0, semA1)
        semB = (semB0, semB1)
        semS = (semS0, semS1)
        pltpu.sync_copy(z_hbm.at[pl.ds(s * _ZR, _ZR)], acc.at[pl.ds(s * _ZR, _ZR)])
        plsc.subcore_barrier()

        def gather(cc, p):
            ga = pltpu.async_copy(A_hbm.at[didx.at[cc]], rowsA.at[p], semA[p])
            gb = pltpu.async_copy(B_hbm.at[sidx.at[cc]], rowsB.at[p], semB[p])
            return ga, gb

        def compute(p):
            def row(r, carry2):
                for kk in range(4):
                    pa = rowsA[p, r, pl.ds(kk * 16, 16)]
                    pb = rowsB[p, r, pl.ds(kk * 16, 16)]
                    alo = lax.bitcast_convert_type(pa << 16, jnp.float32)
                    blo = lax.bitcast_convert_type(pb << 16, jnp.float32)
                    ahi = lax.bitcast_convert_type(pa & jnp.int32(-65536), jnp.float32)
                    bhi = lax.bitcast_convert_type(pb & jnp.int32(-65536), jnp.float32)
                    mlo = alo + blo
                    mhi = ahi + bhi
                    msg[p, r, pl.ds(kk * 16, 16)] = jnp.maximum(mlo, 0.01 * mlo)
                    msg[p, r, pl.ds(64 + kk * 16, 16)] = jnp.maximum(mhi, 0.01 * mhi)
                return carry2
            lax.fori_loop(0, CHW, row, 0)

        def superchunk(q, carry):
            pltpu.sync_copy(src_hbm.at[base + q], sidx)
            pltpu.sync_copy(dst_hbm.at[base + q], didx)
            g = [None, None]
            sc_pend = [None, None]
            g[0] = gather(0, 0)
            for cc in range(8):
                p = cc & 1
                g[p][0].wait()
                g[p][1].wait()
                if cc < 7:
                    g[1 - p] = gather(cc + 1, 1 - p)
                if sc_pend[p] is not None:
                    sc_pend[p].wait()
                    sc_pend[p] = None
                compute(p)
                sc_pend[p] = pltpu.async_copy(
                    msg.at[p], acc.at[didx.at[cc]], semS[p], add=True)
            sc_pend[0].wait()
            sc_pend[1].wait()
            return carry

        lax.fori_loop(0, nq, superchunk, 0)
        plsc.subcore_barrier()
        pltpu.sync_copy(acc.at[pl.ds(s * _ZR, _ZR)],
                        out_hbm.at[c, pl.ds(s * _ZR, _ZR)])

        @pl.when(s == 0)
        def _():
            pltpu.sync_copy(z_hbm.at[pl.ds(0, _NTAIL)],
                            out_hbm.at[c, pl.ds(N_ACC, _NTAIL)])

    return k(A, B, src3, dst3, zeros_big)


def _sc_degree(dst3, ones16, zeros16, nq):
    @functools.partial(
        pl.kernel,
        out_type=jax.ShapeDtypeStruct((2, N_PAD, 16), jnp.float32),
        mesh=_MESH,
        scratch_types=[
            pltpu.VMEM((8, CHW), jnp.int32),
            pltpu.VMEM((CHW, 16), jnp.float32),
            pltpu.VMEM_SHARED((N_ACC, 16), jnp.float32),
            pltpu.SemaphoreType.DMA,
        ],
    )
    def k(dst_hbm, ones_hbm, z_hbm, out_hbm, didx, ones_v, acc, semS):
        c, s, wid = _sc_worker_ids()
        base = wid * nq
        pltpu.sync_copy(ones_hbm, ones_v)
        pltpu.sync_copy(z_hbm.at[pl.ds(s * _ZR, _ZR), pl.ds(0, 16)],
                        acc.at[pl.ds(s * _ZR, _ZR)])
        plsc.subcore_barrier()

        def superchunk(q, carry):
            pltpu.sync_copy(dst_hbm.at[base + q], didx)
            pend = []
            for cc in range(8):
                pend.append(pltpu.async_copy(
                    ones_v, acc.at[didx.at[cc]], semS, add=True))
            for d in pend:
                d.wait()
            return carry

        lax.fori_loop(0, nq, superchunk, 0)
        plsc.subcore_barrier()
        pltpu.sync_copy(acc.at[pl.ds(s * _ZR, _ZR)],
                        out_hbm.at[c, pl.ds(s * _ZR, _ZR)])

        @pl.when(s == 0)
        def _():
            pltpu.sync_copy(z_hbm.at[pl.ds(0, _NTAIL), pl.ds(0, 16)],
                            out_hbm.at[c, pl.ds(N_ACC, _NTAIL)])

    return k(dst3, ones16, zeros16)


def _sc_ptr_score(Xp, Vv, ce0_3d, ce1_3d, theta816, nq):
    @functools.partial(
        pl.kernel,
        out_type=jax.ShapeDtypeStruct((EC_PAD // CHW, CHW, 16), jnp.float32),
        mesh=_MESH,
        compiler_params=pltpu.CompilerParams(use_tc_tiling_on_sc=False),
        scratch_types=[
            pltpu.VMEM((8, CHW), jnp.int32),
            pltpu.VMEM((8, CHW), jnp.int32),
            pltpu.VMEM((2, CHW, H // 2), jnp.int32),
            pltpu.VMEM((2, CHW, H // 2), jnp.int32),
            pltpu.VMEM((2, CHW, 16), jnp.float32),
            pltpu.VMEM((8, 16), jnp.float32),
            pltpu.SemaphoreType.DMA,
            pltpu.SemaphoreType.DMA,
            pltpu.SemaphoreType.DMA,
            pltpu.SemaphoreType.DMA,
            pltpu.SemaphoreType.DMA,
            pltpu.SemaphoreType.DMA,
        ],
    )
    def k(Xp_hbm, Vv_hbm, i0_hbm, i1_hbm, th_hbm, out_hbm,
          i0, i1, rowsX, rowsV, zbuf, thv,
          semA0, semA1, semB0, semB1, semS0, semS1):
        c, s, wid = _sc_worker_ids()
        base = wid * nq
        semA = (semA0, semA1)
        semB = (semB0, semB1)
        semS = (semS0, semS1)
        pltpu.sync_copy(th_hbm, thv)

        def gather(cc, p):
            ga = pltpu.async_copy(Xp_hbm.at[i0.at[cc]], rowsX.at[p], semA[p])
            gb = pltpu.async_copy(Vv_hbm.at[i1.at[cc]], rowsV.at[p], semB[p])
            return ga, gb

        def compute(p):
            th = [thv[kk, :] for kk in range(8)]

            def row(r, carry2):
                acc = jnp.zeros((16,), jnp.float32)
                for kk in range(4):
                    px = rowsX[p, r, pl.ds(kk * 16, 16)]
                    pv = rowsV[p, r, pl.ds(kk * 16, 16)]
                    zlo = (lax.bitcast_convert_type(px << 16, jnp.float32) +
                           lax.bitcast_convert_type(pv << 16, jnp.float32))
                    zhi = (lax.bitcast_convert_type(px & jnp.int32(-65536), jnp.float32) +
                           lax.bitcast_convert_type(pv & jnp.int32(-65536), jnp.float32))
                    tlo = 1.0 - 2.0 / (jnp.exp(2.0 * zlo) + 1.0)
                    thi = 1.0 - 2.0 / (jnp.exp(2.0 * zhi) + 1.0)
                    acc = acc + th[kk] * tlo + th[4 + kk] * thi
                zbuf[p, r, :] = acc
                return carry2
            lax.fori_loop(0, CHW, row, 0)

        def superchunk(q, carry):
            pltpu.sync_copy(i0_hbm.at[base + q], i0)
            pltpu.sync_copy(i1_hbm.at[base + q], i1)
            g = [None, None]
            st_pend = [None, None]
            g[0] = gather(0, 0)
            for cc in range(8):
                p = cc & 1
                g[p][0].wait()
                g[p][1].wait()
                if cc < 7:
                    g[1 - p] = gather(cc + 1, 1 - p)
                if st_pend[p] is not None:
                    st_pend[p].wait()
                    st_pend[p] = None
                compute(p)
                st_pend[p] = pltpu.async_copy(
                    zbuf.at[p], out_hbm.at[(base + q) * 8 + cc], semS[p])
            st_pend[0].wait()
            st_pend[1].wait()
            return carry

        lax.fori_loop(0, nq, superchunk, 0)

    return k(Xp, Vv, ce0_3d, ce1_3d, theta816)


def _sc_ptr_scatter(x, ce0_3d, ce1_3d, y3d, zeros_big, nq):
    @functools.partial(
        pl.kernel,
        out_type=jax.ShapeDtypeStruct((2, N_PAD, H), jnp.float32),
        mesh=_MESH,
        compiler_params=pltpu.CompilerParams(use_tc_tiling_on_sc=False),
        scratch_types=[
            pltpu.VMEM((8, CHW), jnp.int32),
            pltpu.VMEM((8, CHW), jnp.int32),
            pltpu.VMEM((2, CHW, H // 2), jnp.int32),
            pltpu.VMEM((2, CHW, H), jnp.float32),
            pltpu.VMEM((8, CHW), jnp.float32),
            pltpu.VMEM_SHARED((N_ACC, H), jnp.float32),
            pltpu.SemaphoreType.DMA,
            pltpu.SemaphoreType.DMA,
            pltpu.SemaphoreType.DMA,
            pltpu.SemaphoreType.DMA,
        ],
    )
    def k(x_hbm, i0_hbm, i1_hbm, y_hbm, z_hbm, out_hbm,
          i0, i1, rowsX, msg, ybuf, acc, semA0, semA1, semS0, semS1):
        c, s, wid = _sc_worker_ids()
        base = wid * nq
        semA = (semA0, semA1)
        semS = (semS0, semS1)
        pltpu.sync_copy(z_hbm.at[pl.ds(s * _ZR, _ZR)], acc.at[pl.ds(s * _ZR, _ZR)])
        plsc.subcore_barrier()

        def compute(p, cc):
            def grp(gg, carry2):
                yvec = ybuf[cc, pl.ds(gg * 16, 16)]
                for lane in range(16):
                    r = gg * 16 + lane
                    yv = yvec[lane]
                    for kk in range(4):
                        px = rowsX[p, r, pl.ds(kk * 16, 16)]
                        msg[p, r, pl.ds(kk * 16, 16)] = (
                            lax.bitcast_convert_type(px << 16, jnp.float32) * yv)
                        msg[p, r, pl.ds(64 + kk * 16, 16)] = (
                            lax.bitcast_convert_type(px & jnp.int32(-65536), jnp.float32) * yv)
                return carry2
            lax.fori_loop(0, CHW // 16, grp, 0)

        def superchunk(q, carry):
            pltpu.sync_copy(i0_hbm.at[base + q], i0)
            pltpu.sync_copy(i1_hbm.at[base + q], i1)
            pltpu.sync_copy(y_hbm.at[base + q], ybuf)
            g = [None, None]
            sc_pend = [None, None]
            g[0] = pltpu.async_copy(x_hbm.at[i0.at[0]], rowsX.at[0], semA[0])
            for cc in range(8):
                p = cc & 1
                g[p].wait()
                if cc < 7:
                    g[1 - p] = pltpu.async_copy(
                        x_hbm.at[i0.at[cc + 1]], rowsX.at[1 - p], semA[1 - p])
                if sc_pend[p] is not None:
                    sc_pend[p].wait()
                    sc_pend[p] = None
                compute(p, cc)
                sc_pend[p] = pltpu.async_copy(
                    msg.at[p], acc.at[i1.at[cc]], semS[p], add=True)
            sc_pend[0].wait()
            sc_pend[1].wait()
            return carry

        lax.fori_loop(0, nq, superchunk, 0)
        plsc.subcore_barrier()
        pltpu.sync_copy(acc.at[pl.ds(s * _ZR, _ZR)],
                        out_hbm.at[c, pl.ds(s * _ZR, _ZR)])

        @pl.when(s == 0)
        def _():
            pltpu.sync_copy(z_hbm.at[pl.ds(0, _NTAIL)],
                            out_hbm.at[c, pl.ds(N_ACC, _NTAIL)])

    return k(x, ce0_3d, ce1_3d, y3d, zeros_big)


def _pad_rows(a, n):
    return jnp.pad(a, ((0, n - a.shape[0]), (0, 0)))


def _prep_edges(ei, epad):
    e = ei.shape[1]
    p = jnp.pad(ei, ((0, 0), (0, epad - e)), constant_values=DUMMY)
    return (p[0].reshape(epad // SCH, 8, CHW), p[1].reshape(epad // SCH, 8, CHW))


def _pe_host():
    pos = np.arange(100, dtype=np.float32)[:, None]
    i2 = np.arange(0, H, 2, dtype=np.float32)
    ang = pos / (10000.0 ** (i2 / H))
    t = np.zeros((128, H), dtype=np.float32)
    t[:100, 0::2] = np.sin(ang)
    t[:100, 1::2] = np.cos(ang)
    return jnp.asarray(t)


def kernel(local_x, local_edge_index, node_cluster, node_ratio, voxel_x,
           voxel_edge_index, voxel_level, cross_edge_index, program_noise,
           voxel_noise, params):
    nq_p = EP_PAD // NW // SCH
    nq_v = EV_PAD // NW // SCH
    src_p, dst_p = _prep_edges(local_edge_index, EP_PAD)
    src_v, dst_v = _prep_edges(voxel_edge_index, EV_PAD)
    ce0, ce1 = _prep_edges(cross_edge_index, EC_PAD)

    ids3 = jnp.pad(node_cluster, (0, N_PAD - N_REAL),
                   constant_values=NC_PAD - 1).reshape(N_PAD // BLK, 1, BLK)
    lvl3 = jnp.pad(voxel_level, (0, N_PAD - N_REAL)).reshape(N_PAD // BLK, 1, BLK)
    ratio = _pad_rows(jnp.sum(node_ratio, axis=1)[:, None], N_PAD)

    zeros_big = jnp.zeros((N_PAD, H), jnp.float32)
    zeros16 = jnp.zeros((N_PAD, 16), jnp.float32)
    ones16 = jnp.ones((CHW, 16), jnp.float32)
    pe_pad = _pe_host()

    u2d = {}
    for li in (1, 3):
        u = jax.random.uniform(jax.random.fold_in(jax.random.key(42), li),
                               (EC_REAL,), minval=1e-9, maxval=1.0,
                               dtype=jnp.float32)
        u2d[li] = jnp.pad(u, (0, EC_PAD - EC_REAL),
                          constant_values=0.5).reshape(_ZROWS, 128)

    degp = _sc_degree(dst_p, ones16, zeros16, nq_p)
    deg = _deg_reduce(degp)
    ccnt = _cluster_counts(ids3)

    pW, pb = params["p_enc"]["W"], params["p_enc"]["b"]
    x = _fused_rows(N_PAD,
                    [("mm", _pad_rows(local_x, N_PAD), pW[:128]),
                     ("mm", _pad_rows(program_noise, N_PAD), pW[128:])],
                    bias=pb, act="lrelu")
    pos = _fused_rows(N_PAD, [("onehot", lvl3, pe_pad, None)])
    vW, vb = params["v_enc"]["W"], params["v_enc"]["b"]
    v = _fused_rows(N_PAD,
                    [("mm", _pad_rows(voxel_x, N_PAD), vW[:128]),
                     ("mm", _pad_rows(voxel_noise, N_PAD), vW[128:])],
                    bias=vb, act="lrelu", residual=pos)

    for l in range(P_STEPS):
        W, b = params["p_msg"][l]["W"], params["p_msg"][l]["b"]
        A = _fused_rows(N_PAD, [("mm", x, W[:128])], bias=b, pack=True)
        B = _fused_rows(N_PAD, [("mm", x, W[128:])], pack=True)
        csum = _cluster_sum(x, ids3)
        U, ub = params["p_upd"][l]["W"], params["p_upd"][l]["b"]
        CU = _cm_proj(csum, ccnt, U[256:])
        parts = _sc_edge_aggr(A, B, src_p, dst_p, zeros_big, nq_p)
        x = _fused_rows(N_PAD,
                        [("mm", x, U[:128]),
                         ("parts", parts, deg, U[128:256]),
                         ("onehot", ids3, CU, ratio)],
                        bias=ub, act="lrelu", residual=x)

    ptr = params["ptr"]
    Xp = _fused_rows(N_PAD, [("mm", x, ptr["Wp"]["W"])], bias=ptr["Wp"]["b"],
                     pack=True)
    xpk = _pack_rows(x)
    theta816 = ptr["theta"][:, 0].reshape(8, 16)

    for li in range(V_STEPS):
        W, b = params["v_msg"][li]["W"], params["v_msg"][li]["b"]
        A = _fused_rows(N_PAD, [("mm", v, W[:128]), ("mm", pos, W[256:])],
                        bias=b, pack=True)
        B = _fused_rows(N_PAD, [("mm", v, W[128:256]), ("mm", pos, -W[256:])],
                        pack=True)
        parts = _sc_edge_aggr(A, B, src_v, dst_v, zeros_big, nq_v)
        U, ub = params["v_upd"][li]["W"], params["v_upd"][li]["b"]
        v = _fused_rows(N_PAD,
                        [("mm", v, U[:128]), ("parts", parts, None, U[128:])],
                        bias=ub, act="lrelu", residual=v)
        if (li + 1) % 2 == 0:
            Vv = _fused_rows(N_PAD, [("mm", v, ptr["Wv"]["W"])],
                             bias=ptr["Wv"]["b"], pack=True)
            h = _fused_rows(N_PAD, [("mm", v, ptr["m1"]["W"])],
                            bias=ptr["m1"]["b"], act="lrelu")
            mask = _fused_rows(N_PAD, [("mm", h, ptr["m2"]["W"])],
                               bias=ptr["m2"]["b"], act="sigmoid", out_dim=1)
            zp = _sc_ptr_score(Xp, Vv, ce0, ce1, theta816, nq_v)
            z2d, m = _ptr_phase_a(zp.reshape(EC_PAD, 16), u2d[li])
            y2d, S = _ptr_phase_b(z2d, m)
            y3d = y2d.reshape(EC_PAD // SCH, 8, CHW)
            parts = _sc_ptr_scatter(xpk, ce0, ce1, y3d, zeros_big, nq_v)
            v = _fused_rows(N_PAD, [("gparts", parts, mask, S)], residual=v)

    return v[:N_REAL]
